# baseline (speedup 1.0000x reference)
import numpy as np
import jax
import jax.numpy as jnp
from jax.experimental import pallas as pl
from jax.experimental.pallas import tpu as pltpu

ALPHA = 4


def _slow_indices(T: int):
    return [int(v) for v in np.linspace(0, T - 1, T // ALPHA).astype(np.int64)]


def _pack_body(sel_ref, pos_ref, src_ref, slow_ref, fast_ref):
    t = pl.program_id(1)
    fast_ref[...] = src_ref[...]

    @pl.when(sel_ref[t] == 1)
    def _():
        slow_ref[...] = src_ref[...]


def kernel(frames):
    C, T, H, W = frames.shape
    idx = _slow_indices(T)
    S = len(idx)
    sel = np.zeros((T,), dtype=np.int32)
    sel[idx] = 1
    pos = np.maximum(np.cumsum(sel) - 1, 0).astype(np.int32)

    grid_spec = pltpu.PrefetchScalarGridSpec(
        num_scalar_prefetch=2,
        grid=(C, T),
        in_specs=[
            pl.BlockSpec((1, 1, H, W), lambda c, t, sel_ref, pos_ref: (c, t, 0, 0)),
        ],
        out_specs=[
            pl.BlockSpec(
                (1, 1, H, W), lambda c, t, sel_ref, pos_ref: (c, pos_ref[t], 0, 0)
            ),
            pl.BlockSpec((1, 1, H, W), lambda c, t, sel_ref, pos_ref: (c, t, 0, 0)),
        ],
    )

    slow, fast = pl.pallas_call(
        _pack_body,
        grid_spec=grid_spec,
        out_shape=[
            jax.ShapeDtypeStruct((C, S, H, W), frames.dtype),
            jax.ShapeDtypeStruct((C, T, H, W), frames.dtype),
        ],
    )(jnp.asarray(sel), jnp.asarray(pos), frames)

    return (slow, fast)
